# baseline (speedup 1.0000x reference)
import functools

import jax
import jax.numpy as jnp
from jax import lax
from jax.experimental import pallas as pl
from jax.experimental.pallas import tpu as pltpu
from jax.experimental.pallas import tpu_sc as plsc

N = 10000
E = 320000
EL = 20000
D_IN = 128
D_H = 128
D_OUT = 64

NC, NS, L = 2, 16, 16
NW = NC * NS
NP = 10240
RPT = NP // NS
C = 128
CS = 10
SUP = 8
EP = NW * SUP * CS * C
ELP = 20480
DKC = 5

_mesh = plsc.VectorSubcoreMesh(core_axis_name="c", subcore_axis_name="s")


def _wid():
    return lax.axis_index("s") * NC + lax.axis_index("c")


@functools.partial(
    pl.kernel,
    out_type=jax.ShapeDtypeStruct((NC, NP), jnp.float32),
    mesh=_mesh,
    scratch_types=[
        pltpu.VMEM((CS, C), jnp.int32),
        pltpu.VMEM((C,), jnp.float32),
        pltpu.VMEM((RPT,), jnp.float32),
        pltpu.VMEM_SHARED((NP,), jnp.float32),
    ],
)
def _sc_degree(dst_hbm, deg_out, dst_v, ones_v, zb_v, acc_sh):
    c = lax.axis_index("c")
    s = lax.axis_index("s")
    w = _wid()

    def fill(i, _):
        ones_v[pl.ds(i * L, L)] = jnp.ones((L,), jnp.float32)
        return 0

    lax.fori_loop(0, C // L, fill, 0)

    def zfill(i, _):
        zb_v[pl.ds(i * L, L)] = jnp.zeros((L,), jnp.float32)
        return 0

    lax.fori_loop(0, RPT // L, zfill, 0)
    pltpu.sync_copy(zb_v, acc_sh.at[pl.ds(s * RPT, RPT)])
    plsc.subcore_barrier()

    def sup(g, _):
        pltpu.sync_copy(dst_hbm.at[w, g], dst_v)
        for j in range(CS):
            pltpu.sync_copy(ones_v, acc_sh.at[dst_v.at[j]], add=True)
        return 0

    lax.fori_loop(0, SUP, sup, 0)
    plsc.subcore_barrier()
    pltpu.sync_copy(acc_sh.at[pl.ds(s * RPT, RPT)],
                    deg_out.at[c, pl.ds(s * RPT, RPT)])


def _make_sc_agg(D):
    @functools.partial(
        pl.kernel,
        out_type=jax.ShapeDtypeStruct((NC, NP, D), jnp.float32),
        mesh=_mesh,
        scratch_types=[
            pltpu.VMEM((CS, C), jnp.int32),
            pltpu.VMEM((CS, C), jnp.int32),
            pltpu.VMEM((C, D), jnp.float32),
            pltpu.VMEM((C, D), jnp.float32),
            pltpu.VMEM_SHARED((NP, D), jnp.float32),
            pltpu.SemaphoreType.DMA,
            pltpu.SemaphoreType.DMA,
            pltpu.SemaphoreType.DMA,
            pltpu.SemaphoreType.DMA,
        ],
    )
    def agg(hp_hbm, src_hbm, dst_hbm, out_hbm,
            src_v, dst_v, rows0_v, rows1_v, acc_sh,
            semg0, semg1, sems0, sems1):
        c = lax.axis_index("c")
        s = lax.axis_index("s")
        w = _wid()

        zv = jnp.zeros((L,), jnp.float32)

        def zfill(i, _):
            rows0_v[i // (D // L), pl.ds((i % (D // L)) * L, L)] = zv
            return 0

        lax.fori_loop(0, C * (D // L), zfill, 0)
        for j in range(RPT // C):
            pltpu.sync_copy(rows0_v, acc_sh.at[pl.ds(s * RPT + j * C, C), :])
        plsc.subcore_barrier()

        bufs = (rows0_v, rows1_v)
        gsems = (semg0, semg1)
        ssems = (sems0, sems1)

        def sup(g, _):
            pltpu.sync_copy(src_hbm.at[w, g], src_v)
            pltpu.sync_copy(dst_hbm.at[w, g], dst_v)
            sd = [None] * CS
            for k in range(CS // 2):
                j0 = 2 * k
                j1 = j0 + 1
                if k > 0:
                    sd[j0 - 2].wait()
                    sd[j0 - 1].wait()
                g0 = pltpu.async_copy(hp_hbm.at[src_v.at[j0]], bufs[0],
                                      gsems[0])
                g1 = pltpu.async_copy(hp_hbm.at[src_v.at[j1]], bufs[1],
                                      gsems[1])
                g0.wait()
                sd[j0] = pltpu.async_copy(
                    bufs[0], acc_sh.at[dst_v.at[j0]], ssems[0], add=True)
                g1.wait()
                sd[j1] = pltpu.async_copy(
                    bufs[1], acc_sh.at[dst_v.at[j1]], ssems[1], add=True)
            sd[CS - 2].wait()
            sd[CS - 1].wait()
            return 0

        lax.fori_loop(0, SUP, sup, 0)
        plsc.subcore_barrier()
        pltpu.sync_copy(acc_sh.at[pl.ds(s * RPT, RPT), :],
                        out_hbm.at[c, pl.ds(s * RPT, RPT), :])

    return agg


_sc_agg_128 = _make_sc_agg(D_H)


@functools.partial(
    pl.kernel,
    out_type=(jax.ShapeDtypeStruct((ELP, D_H), jnp.float32),
              jax.ShapeDtypeStruct((ELP, D_H), jnp.float32)),
    mesh=_mesh,
    scratch_types=[
        pltpu.VMEM((DKC, C), jnp.int32),
        pltpu.VMEM((DKC, C), jnp.int32),
        [pltpu.VMEM((C, D_H), jnp.float32)] * 4,
        [pltpu.SemaphoreType.DMA] * 4,
        [pltpu.SemaphoreType.DMA] * 4,
    ],
)
def _sc_decode(z_hbm, sidx_hbm, didx_hbm, zs_out, zd_out,
               sidx_v, didx_v, bufs, gsems, csems):
    w = _wid()
    pltpu.sync_copy(sidx_hbm.at[w], sidx_v)
    pltpu.sync_copy(didx_hbm.at[w], didx_v)

    gd = [None] * (DKC + 1)
    cd = [None] * DKC

    def gather(j, p):
        return (pltpu.async_copy(z_hbm.at[sidx_v.at[j]], bufs[2 * p], gsems[2 * p]),
                pltpu.async_copy(z_hbm.at[didx_v.at[j]], bufs[2 * p + 1], gsems[2 * p + 1]))

    gd[0] = gather(0, 0)
    for j in range(DKC):
        p = j % 2
        q = 1 - p
        if j >= 1:
            cd[j - 1][0].wait()
            cd[j - 1][1].wait()
        if j + 1 < DKC:
            gd[j + 1] = gather(j + 1, q)
        base = w * (DKC * C) + j * C
        gd[j][0].wait()
        gd[j][1].wait()
        cd[j] = (pltpu.async_copy(bufs[2 * p], zs_out.at[pl.ds(base, C), :], csems[2 * p]),
                 pltpu.async_copy(bufs[2 * p + 1], zd_out.at[pl.ds(base, C), :], csems[2 * p + 1]))
    cd[DKC - 1][0].wait()
    cd[DKC - 1][1].wait()


_BR = 1024


def _dinv_block(d0_blk, d1_blk):
    d = d0_blk + d1_blk + 1.0
    return jax.lax.rsqrt(d)[:, :1]


def _tc1_body(x_ref, w_ref, d0_ref, d1_ref, hp_ref):
    h = jnp.dot(x_ref[...], w_ref[...], preferred_element_type=jnp.float32)
    hp_ref[...] = h * _dinv_block(d0_ref[...], d1_ref[...])


def _tc2_body(a0_ref, a1_ref, hp_ref, d0_ref, d1_ref, b_ref, w_ref, hp2_ref):
    dinv = _dinv_block(d0_ref[...], d1_ref[...])
    z1 = jnp.maximum(
        dinv * (a0_ref[...] + a1_ref[...] + hp_ref[...]) + b_ref[...], 0.0)
    h2 = jnp.dot(z1, w_ref[...], preferred_element_type=jnp.float32)
    hp2_ref[...] = h2 * dinv


def _tc3_body(a0_ref, a1_ref, hp2_ref, d0_ref, d1_ref, b_ref, z_ref):
    dinv = _dinv_block(d0_ref[...], d1_ref[...])
    z_ref[...] = dinv * (a0_ref[...] + a1_ref[...] + hp2_ref[...]) + b_ref[...]


def _tc4_body(zs_ref, zd_ref, o_ref):
    p = (zs_ref[...] * zd_ref[...]).reshape(_BR // 128, 128, D_H)
    o_ref[...] = jnp.sum(p, axis=-1)


def _row_spec(d):
    return pl.BlockSpec((_BR, d), lambda i: (i, 0))


_deg_spec = pl.BlockSpec((_BR, 8), lambda i: (i, 0))


def _full(shape):
    return pl.BlockSpec(shape, lambda i: (0,) * len(shape))


_GRID = NP // _BR


def _tc1(x_p, W1, d0, d1):
    return pl.pallas_call(
        _tc1_body,
        grid=(_GRID,),
        in_specs=[_row_spec(D_IN), _full((D_IN, D_H)), _deg_spec, _deg_spec],
        out_specs=_row_spec(D_H),
        out_shape=jax.ShapeDtypeStruct((NP, D_H), jnp.float32),
    )(x_p, W1, d0, d1)


def _tc2(a0, a1, hp, d0, d1, b1, W2):
    return pl.pallas_call(
        _tc2_body,
        grid=(_GRID,),
        in_specs=[_row_spec(D_H), _row_spec(D_H), _row_spec(D_H), _deg_spec,
                  _deg_spec, _full((1, D_H)), _full((D_H, D_H))],
        out_specs=_row_spec(D_H),
        out_shape=jax.ShapeDtypeStruct((NP, D_H), jnp.float32),
    )(a0, a1, hp, d0, d1, b1, W2)


def _tc3(a0, a1, hp2, d0, d1, b2):
    return pl.pallas_call(
        _tc3_body,
        grid=(_GRID,),
        in_specs=[_row_spec(D_H), _row_spec(D_H), _row_spec(D_H),
                  _deg_spec, _deg_spec, _full((1, D_H))],
        out_specs=_row_spec(D_H),
        out_shape=jax.ShapeDtypeStruct((NP, D_H), jnp.float32),
    )(a0, a1, hp2, d0, d1, b2)


def _tc4(zs, zd):
    return pl.pallas_call(
        _tc4_body,
        grid=(ELP // _BR,),
        in_specs=[_row_spec(D_H), _row_spec(D_H)],
        out_specs=pl.BlockSpec((_BR // 128, 128), lambda i: (i, 0)),
        out_shape=jax.ShapeDtypeStruct((ELP // 128, 128), jnp.float32),
    )(zs, zd)


def kernel(x, edge_index, edge_label_index, W1, b1, W2, b2):
    x_p = jnp.pad(x, ((0, NP - N), (0, 0)))
    src = jnp.concatenate(
        [edge_index[0], jnp.zeros((EP - E,), jnp.int32)]).reshape(NW, SUP, CS, C)
    dst = jnp.concatenate(
        [edge_index[1], jnp.full((EP - E,), N, jnp.int32)]).reshape(NW, SUP, CS, C)
    pad = jnp.zeros((ELP - EL,), jnp.int32)
    sidx = jnp.concatenate([edge_label_index[0], pad]).reshape(NW, DKC, C)
    didx = jnp.concatenate([edge_label_index[1], pad]).reshape(NW, DKC, C)

    deg2 = _sc_degree(dst)
    d0 = jnp.broadcast_to(deg2[0][:, None], (NP, 8))
    d1 = jnp.broadcast_to(deg2[1][:, None], (NP, 8))

    W2p = jnp.pad(W2, ((0, 0), (0, D_H - D_OUT)))
    b2p = jnp.pad(b2, (0, D_H - D_OUT))

    hp = _tc1(x_p, W1, d0, d1)
    acc1 = _sc_agg_128(hp, src, dst)
    hp2 = _tc2(acc1[0], acc1[1], hp, d0, d1, b1.reshape(1, D_H), W2p)
    acc2 = _sc_agg_128(hp2, src, dst)
    z = _tc3(acc2[0], acc2[1], hp2, d0, d1, b2p.reshape(1, D_H))

    zs, zd = _sc_decode(z, sidx, didx)
    dots = _tc4(zs, zd)
    return dots.reshape(ELP)[:EL]


# speedup vs baseline: 1.0464x; 1.0464x over previous
import functools

import jax
import jax.numpy as jnp
from jax import lax
from jax.experimental import pallas as pl
from jax.experimental.pallas import tpu as pltpu
from jax.experimental.pallas import tpu_sc as plsc

N = 10000
E = 320000
EL = 20000
D_IN = 128
D_H = 128
D_OUT = 64

NC, NS, L = 2, 16, 16
NW = NC * NS
NP = 10240
RPT = NP // NS
C = 128
CS = 10
SUP = 8
EP = NW * SUP * CS * C
ELP = 20480
DKC = 5

_mesh = plsc.VectorSubcoreMesh(core_axis_name="c", subcore_axis_name="s")


def _wid():
    return lax.axis_index("s") * NC + lax.axis_index("c")


@functools.partial(
    pl.kernel,
    out_type=jax.ShapeDtypeStruct((NC, NP), jnp.float32),
    mesh=_mesh,
    scratch_types=[
        pltpu.VMEM((CS, C), jnp.int32),
        pltpu.VMEM((C,), jnp.float32),
        pltpu.VMEM((RPT,), jnp.float32),
        pltpu.VMEM_SHARED((NP,), jnp.float32),
    ],
)
def _sc_degree(dst_hbm, deg_out, dst_v, ones_v, zb_v, acc_sh):
    c = lax.axis_index("c")
    s = lax.axis_index("s")
    w = _wid()

    def fill(i, _):
        ones_v[pl.ds(i * L, L)] = jnp.ones((L,), jnp.float32)
        return 0

    lax.fori_loop(0, C // L, fill, 0)

    def zfill(i, _):
        zb_v[pl.ds(i * L, L)] = jnp.zeros((L,), jnp.float32)
        return 0

    lax.fori_loop(0, RPT // L, zfill, 0)
    pltpu.sync_copy(zb_v, acc_sh.at[pl.ds(s * RPT, RPT)])
    plsc.subcore_barrier()

    def sup(g, _):
        pltpu.sync_copy(dst_hbm.at[w, g], dst_v)
        for j in range(CS):
            pltpu.sync_copy(ones_v, acc_sh.at[dst_v.at[j]], add=True)
        return 0

    lax.fori_loop(0, SUP, sup, 0)
    plsc.subcore_barrier()
    pltpu.sync_copy(acc_sh.at[pl.ds(s * RPT, RPT)],
                    deg_out.at[c, pl.ds(s * RPT, RPT)])


def _make_sc_agg(D):
    @functools.partial(
        pl.kernel,
        out_type=jax.ShapeDtypeStruct((NC, NP, D), jnp.float32),
        mesh=_mesh,
        scratch_types=[
            pltpu.VMEM((CS, C), jnp.int32),
            pltpu.VMEM((CS, C), jnp.int32),
            pltpu.VMEM((C, D), jnp.float32),
            pltpu.VMEM((C, D), jnp.float32),
            pltpu.VMEM_SHARED((NP, D), jnp.float32),
            pltpu.SemaphoreType.DMA,
            pltpu.SemaphoreType.DMA,
            pltpu.SemaphoreType.DMA,
            pltpu.SemaphoreType.DMA,
        ],
    )
    def agg(hp_hbm, src_hbm, dst_hbm, out_hbm,
            src_v, dst_v, rows0_v, rows1_v, acc_sh,
            semg0, semg1, sems0, sems1):
        c = lax.axis_index("c")
        s = lax.axis_index("s")
        w = _wid()

        zv = jnp.zeros((L,), jnp.float32)

        def zfill(i, _):
            rows0_v[i // (D // L), pl.ds((i % (D // L)) * L, L)] = zv
            return 0

        lax.fori_loop(0, C * (D // L), zfill, 0)
        for j in range(RPT // C):
            pltpu.sync_copy(rows0_v, acc_sh.at[pl.ds(s * RPT + j * C, C), :])
        plsc.subcore_barrier()

        bufs = (rows0_v, rows1_v)
        gsems = (semg0, semg1)
        ssems = (sems0, sems1)

        def sup(g, _):
            pltpu.sync_copy(src_hbm.at[w, g], src_v)
            pltpu.sync_copy(dst_hbm.at[w, g], dst_v)
            gd = [None] * CS
            sd = [None] * CS
            gd[0] = pltpu.async_copy(hp_hbm.at[src_v.at[0]], bufs[0], gsems[0])
            for j in range(CS):
                p = j % 2
                q = 1 - p
                gd[j].wait()
                if j >= 1:
                    sd[j - 1].wait()
                if j < CS - 1:
                    gd[j + 1] = pltpu.async_copy(
                        hp_hbm.at[src_v.at[j + 1]], bufs[q], gsems[q])
                sd[j] = pltpu.async_copy(
                    bufs[p], acc_sh.at[dst_v.at[j]], ssems[p], add=True)
            sd[CS - 1].wait()
            return 0

        lax.fori_loop(0, SUP, sup, 0)
        plsc.subcore_barrier()
        pltpu.sync_copy(acc_sh.at[pl.ds(s * RPT, RPT), :],
                        out_hbm.at[c, pl.ds(s * RPT, RPT), :])

    return agg


_sc_agg_128 = _make_sc_agg(D_H)


@functools.partial(
    pl.kernel,
    out_type=(jax.ShapeDtypeStruct((ELP, D_H), jnp.float32),
              jax.ShapeDtypeStruct((ELP, D_H), jnp.float32)),
    mesh=_mesh,
    scratch_types=[
        pltpu.VMEM((DKC, C), jnp.int32),
        pltpu.VMEM((DKC, C), jnp.int32),
        [pltpu.VMEM((C, D_H), jnp.float32)] * 4,
        [pltpu.SemaphoreType.DMA] * 4,
        [pltpu.SemaphoreType.DMA] * 4,
    ],
)
def _sc_decode(z_hbm, sidx_hbm, didx_hbm, zs_out, zd_out,
               sidx_v, didx_v, bufs, gsems, csems):
    w = _wid()
    pltpu.sync_copy(sidx_hbm.at[w], sidx_v)
    pltpu.sync_copy(didx_hbm.at[w], didx_v)

    gd = [None] * (DKC + 1)
    cd = [None] * DKC

    def gather(j, p):
        return (pltpu.async_copy(z_hbm.at[sidx_v.at[j]], bufs[2 * p], gsems[2 * p]),
                pltpu.async_copy(z_hbm.at[didx_v.at[j]], bufs[2 * p + 1], gsems[2 * p + 1]))

    gd[0] = gather(0, 0)
    for j in range(DKC):
        p = j % 2
        q = 1 - p
        if j >= 1:
            cd[j - 1][0].wait()
            cd[j - 1][1].wait()
        if j + 1 < DKC:
            gd[j + 1] = gather(j + 1, q)
        base = w * (DKC * C) + j * C
        gd[j][0].wait()
        gd[j][1].wait()
        cd[j] = (pltpu.async_copy(bufs[2 * p], zs_out.at[pl.ds(base, C), :], csems[2 * p]),
                 pltpu.async_copy(bufs[2 * p + 1], zd_out.at[pl.ds(base, C), :], csems[2 * p + 1]))
    cd[DKC - 1][0].wait()
    cd[DKC - 1][1].wait()


_BR = 1024


def _dinv_block(d0_blk, d1_blk):
    d = d0_blk + d1_blk + 1.0
    return jax.lax.rsqrt(d)[:, :1]


def _tc1_body(x_ref, w_ref, d0_ref, d1_ref, hp_ref):
    h = jnp.dot(x_ref[...], w_ref[...], preferred_element_type=jnp.float32)
    hp_ref[...] = h * _dinv_block(d0_ref[...], d1_ref[...])


def _tc2_body(a0_ref, a1_ref, hp_ref, d0_ref, d1_ref, b_ref, w_ref, hp2_ref):
    dinv = _dinv_block(d0_ref[...], d1_ref[...])
    z1 = jnp.maximum(
        dinv * (a0_ref[...] + a1_ref[...] + hp_ref[...]) + b_ref[...], 0.0)
    h2 = jnp.dot(z1, w_ref[...], preferred_element_type=jnp.float32)
    hp2_ref[...] = h2 * dinv


def _tc3_body(a0_ref, a1_ref, hp2_ref, d0_ref, d1_ref, b_ref, z_ref):
    dinv = _dinv_block(d0_ref[...], d1_ref[...])
    z_ref[...] = dinv * (a0_ref[...] + a1_ref[...] + hp2_ref[...]) + b_ref[...]


def _tc4_body(zs_ref, zd_ref, o_ref):
    p = (zs_ref[...] * zd_ref[...]).reshape(_BR // 128, 128, D_H)
    o_ref[...] = jnp.sum(p, axis=-1)


def _row_spec(d):
    return pl.BlockSpec((_BR, d), lambda i: (i, 0))


_deg_spec = pl.BlockSpec((_BR, 8), lambda i: (i, 0))


def _full(shape):
    return pl.BlockSpec(shape, lambda i: (0,) * len(shape))


_GRID = NP // _BR


def _tc1(x_p, W1, d0, d1):
    return pl.pallas_call(
        _tc1_body,
        grid=(_GRID,),
        in_specs=[_row_spec(D_IN), _full((D_IN, D_H)), _deg_spec, _deg_spec],
        out_specs=_row_spec(D_H),
        out_shape=jax.ShapeDtypeStruct((NP, D_H), jnp.float32),
    )(x_p, W1, d0, d1)


def _tc2(a0, a1, hp, d0, d1, b1, W2):
    return pl.pallas_call(
        _tc2_body,
        grid=(_GRID,),
        in_specs=[_row_spec(D_H), _row_spec(D_H), _row_spec(D_H), _deg_spec,
                  _deg_spec, _full((1, D_H)), _full((D_H, D_H))],
        out_specs=_row_spec(D_H),
        out_shape=jax.ShapeDtypeStruct((NP, D_H), jnp.float32),
    )(a0, a1, hp, d0, d1, b1, W2)


def _tc3(a0, a1, hp2, d0, d1, b2):
    return pl.pallas_call(
        _tc3_body,
        grid=(_GRID,),
        in_specs=[_row_spec(D_H), _row_spec(D_H), _row_spec(D_H),
                  _deg_spec, _deg_spec, _full((1, D_H))],
        out_specs=_row_spec(D_H),
        out_shape=jax.ShapeDtypeStruct((NP, D_H), jnp.float32),
    )(a0, a1, hp2, d0, d1, b2)


def _tc4(zs, zd):
    return pl.pallas_call(
        _tc4_body,
        grid=(ELP // _BR,),
        in_specs=[_row_spec(D_H), _row_spec(D_H)],
        out_specs=pl.BlockSpec((_BR // 128, 128), lambda i: (i, 0)),
        out_shape=jax.ShapeDtypeStruct((ELP // 128, 128), jnp.float32),
    )(zs, zd)


def kernel(x, edge_index, edge_label_index, W1, b1, W2, b2):
    x_p = jnp.pad(x, ((0, NP - N), (0, 0)))
    src = jnp.concatenate(
        [edge_index[0], jnp.zeros((EP - E,), jnp.int32)]).reshape(NW, SUP, CS, C)
    dst = jnp.concatenate(
        [edge_index[1], jnp.full((EP - E,), N, jnp.int32)]).reshape(NW, SUP, CS, C)
    pad = jnp.zeros((ELP - EL,), jnp.int32)
    sidx = jnp.concatenate([edge_label_index[0], pad]).reshape(NW, DKC, C)
    didx = jnp.concatenate([edge_label_index[1], pad]).reshape(NW, DKC, C)

    deg2 = _sc_degree(dst)
    d0 = jnp.broadcast_to(deg2[0][:, None], (NP, 8))
    d1 = jnp.broadcast_to(deg2[1][:, None], (NP, 8))

    W2p = jnp.pad(W2, ((0, 0), (0, D_H - D_OUT)))
    b2p = jnp.pad(b2, (0, D_H - D_OUT))

    hp = _tc1(x_p, W1, d0, d1)
    acc1 = _sc_agg_128(hp, src, dst)
    hp2 = _tc2(acc1[0], acc1[1], hp, d0, d1, b1.reshape(1, D_H), W2p)
    acc2 = _sc_agg_128(hp2, src, dst)
    z = _tc3(acc2[0], acc2[1], hp2, d0, d1, b2p.reshape(1, D_H))

    zs, zd = _sc_decode(z, sidx, didx)
    dots = _tc4(zs, zd)
    return dots.reshape(ELP)[:EL]


# speedup vs baseline: 1.0863x; 1.0381x over previous
import functools

import jax
import jax.numpy as jnp
from jax import lax
from jax.experimental import pallas as pl
from jax.experimental.pallas import tpu as pltpu
from jax.experimental.pallas import tpu_sc as plsc

N = 10000
E = 320000
EL = 20000
D_IN = 128
D_H = 128
D_OUT = 64

NC, NS, L = 2, 16, 16
NW = NC * NS
NP = 10240
RPT = NP // NS
C = 128
CS = 10
SUP = 8
EP = NW * SUP * CS * C
ELP = 20480
DKC = 5

_mesh = plsc.VectorSubcoreMesh(core_axis_name="c", subcore_axis_name="s")


def _wid():
    return lax.axis_index("s") * NC + lax.axis_index("c")


@functools.partial(
    pl.kernel,
    out_type=jax.ShapeDtypeStruct((NC, NP), jnp.float32),
    mesh=_mesh,
    scratch_types=[
        pltpu.VMEM((CS, C), jnp.int32),
        pltpu.VMEM((C,), jnp.float32),
        pltpu.VMEM((RPT,), jnp.float32),
        pltpu.VMEM_SHARED((NP,), jnp.float32),
    ],
)
def _sc_degree(dst_hbm, deg_out, dst_v, ones_v, zb_v, acc_sh):
    c = lax.axis_index("c")
    s = lax.axis_index("s")
    w = _wid()

    def fill(i, _):
        ones_v[pl.ds(i * L, L)] = jnp.ones((L,), jnp.float32)
        return 0

    lax.fori_loop(0, C // L, fill, 0)

    def zfill(i, _):
        zb_v[pl.ds(i * L, L)] = jnp.zeros((L,), jnp.float32)
        return 0

    lax.fori_loop(0, RPT // L, zfill, 0)
    pltpu.sync_copy(zb_v, acc_sh.at[pl.ds(s * RPT, RPT)])
    plsc.subcore_barrier()

    def sup(g, _):
        pltpu.sync_copy(dst_hbm.at[w, g], dst_v)
        for j in range(CS):
            pltpu.sync_copy(ones_v, acc_sh.at[dst_v.at[j]], add=True)
        return 0

    lax.fori_loop(0, SUP, sup, 0)
    plsc.subcore_barrier()
    pltpu.sync_copy(acc_sh.at[pl.ds(s * RPT, RPT)],
                    deg_out.at[c, pl.ds(s * RPT, RPT)])


def _make_sc_agg(D):
    @functools.partial(
        pl.kernel,
        out_type=jax.ShapeDtypeStruct((NC, NP, D), jnp.float32),
        mesh=_mesh,
        scratch_types=[
            pltpu.VMEM((CS, C), jnp.int32),
            pltpu.VMEM((CS, C), jnp.int32),
            pltpu.VMEM((C, D), jnp.float32),
            pltpu.VMEM((C, D), jnp.float32),
            pltpu.VMEM_SHARED((NP, D), jnp.float32),
            pltpu.SemaphoreType.DMA,
            pltpu.SemaphoreType.DMA,
            pltpu.SemaphoreType.DMA,
            pltpu.SemaphoreType.DMA,
        ],
    )
    def agg(hp_hbm, src_hbm, dst_hbm, out_hbm,
            src_v, dst_v, rows0_v, rows1_v, acc_sh,
            semg0, semg1, sems0, sems1):
        c = lax.axis_index("c")
        s = lax.axis_index("s")
        w = _wid()

        zv = jnp.zeros((L,), jnp.float32)

        def zfill(i, _):
            rows0_v[i // (D // L), pl.ds((i % (D // L)) * L, L)] = zv
            return 0

        lax.fori_loop(0, C * (D // L), zfill, 0)
        for j in range(RPT // C):
            pltpu.sync_copy(rows0_v, acc_sh.at[pl.ds(s * RPT + j * C, C), :])
        plsc.subcore_barrier()

        bufs = (rows0_v, rows1_v)
        gsems = (semg0, semg1)
        ssems = (sems0, sems1)

        def sup(g, _):
            pltpu.sync_copy(src_hbm.at[w, g], src_v)
            pltpu.sync_copy(dst_hbm.at[w, g], dst_v)
            gd = [None] * CS
            sd = [None] * CS
            gd[0] = pltpu.async_copy(hp_hbm.at[src_v.at[0]], bufs[0], gsems[0])
            for j in range(CS):
                p = j % 2
                q = 1 - p
                gd[j].wait()
                if j >= 1:
                    sd[j - 1].wait()
                if j < CS - 1:
                    gd[j + 1] = pltpu.async_copy(
                        hp_hbm.at[src_v.at[j + 1]], bufs[q], gsems[q])
                sd[j] = pltpu.async_copy(
                    bufs[p], acc_sh.at[dst_v.at[j]], ssems[p], add=True)
            sd[CS - 1].wait()
            return 0

        lax.fori_loop(0, SUP, sup, 0)
        plsc.subcore_barrier()
        pltpu.sync_copy(acc_sh.at[pl.ds(s * RPT, RPT), :],
                        out_hbm.at[c, pl.ds(s * RPT, RPT), :])

    return agg


_sc_agg_128 = _make_sc_agg(D_H)


def _make_sc_agg_untiled(D):
    @functools.partial(
        pl.kernel,
        out_type=jax.ShapeDtypeStruct((NC, NP, D), jnp.float32),
        mesh=_mesh,
        compiler_params=pltpu.CompilerParams(use_tc_tiling_on_sc=False),
        scratch_types=[
            pltpu.VMEM((CS, C), jnp.int32),
            pltpu.VMEM((CS, C), jnp.int32),
            pltpu.VMEM((C, D), jnp.float32),
            pltpu.VMEM((C, D), jnp.float32),
            pltpu.VMEM_SHARED((NP, D), jnp.float32),
            pltpu.SemaphoreType.DMA,
            pltpu.SemaphoreType.DMA,
            pltpu.SemaphoreType.DMA,
            pltpu.SemaphoreType.DMA,
        ],
    )
    def agg(hp_hbm, src_hbm, dst_hbm, out_hbm,
            src_v, dst_v, rows0_v, rows1_v, acc_sh,
            semg0, semg1, sems0, sems1):
        c = lax.axis_index("c")
        s = lax.axis_index("s")
        w = _wid()

        zv = jnp.zeros((L,), jnp.float32)

        def zfill(i, _):
            rows0_v[i // (D // L), pl.ds((i % (D // L)) * L, L)] = zv
            return 0

        lax.fori_loop(0, C * (D // L), zfill, 0)
        for j in range(RPT // C):
            pltpu.sync_copy(rows0_v, acc_sh.at[pl.ds(s * RPT + j * C, C), :])
        plsc.subcore_barrier()

        bufs = (rows0_v, rows1_v)
        gsems = (semg0, semg1)
        ssems = (sems0, sems1)

        def sup(g, _):
            pltpu.sync_copy(src_hbm.at[w, g], src_v)
            pltpu.sync_copy(dst_hbm.at[w, g], dst_v)
            gd = [None] * CS
            sd = [None] * CS
            gd[0] = pltpu.async_copy(hp_hbm.at[src_v.at[0]], bufs[0], gsems[0])
            for j in range(CS):
                p = j % 2
                q = 1 - p
                gd[j].wait()
                if j >= 1:
                    sd[j - 1].wait()
                if j < CS - 1:
                    gd[j + 1] = pltpu.async_copy(
                        hp_hbm.at[src_v.at[j + 1]], bufs[q], gsems[q])
                sd[j] = pltpu.async_copy(
                    bufs[p], acc_sh.at[dst_v.at[j]], ssems[p], add=True)
            sd[CS - 1].wait()
            return 0

        lax.fori_loop(0, SUP, sup, 0)
        plsc.subcore_barrier()
        pltpu.sync_copy(acc_sh.at[pl.ds(s * RPT, RPT), :],
                        out_hbm.at[c, pl.ds(s * RPT, RPT), :])

    return agg


_sc_agg_64u = _make_sc_agg_untiled(D_OUT)


@functools.partial(
    pl.kernel,
    out_type=(jax.ShapeDtypeStruct((ELP, D_OUT), jnp.float32),
              jax.ShapeDtypeStruct((ELP, D_OUT), jnp.float32)),
    mesh=_mesh,
    compiler_params=pltpu.CompilerParams(use_tc_tiling_on_sc=False),
    scratch_types=[
        pltpu.VMEM((DKC, C), jnp.int32),
        pltpu.VMEM((DKC, C), jnp.int32),
        [pltpu.VMEM((C, D_OUT), jnp.float32)] * 4,
        [pltpu.SemaphoreType.DMA] * 4,
        [pltpu.SemaphoreType.DMA] * 4,
    ],
)
def _sc_decode(z_hbm, sidx_hbm, didx_hbm, zs_out, zd_out,
               sidx_v, didx_v, bufs, gsems, csems):
    w = _wid()
    pltpu.sync_copy(sidx_hbm.at[w], sidx_v)
    pltpu.sync_copy(didx_hbm.at[w], didx_v)

    gd = [None] * (DKC + 1)
    cd = [None] * DKC

    def gather(j, p):
        return (pltpu.async_copy(z_hbm.at[sidx_v.at[j]], bufs[2 * p], gsems[2 * p]),
                pltpu.async_copy(z_hbm.at[didx_v.at[j]], bufs[2 * p + 1], gsems[2 * p + 1]))

    gd[0] = gather(0, 0)
    for j in range(DKC):
        p = j % 2
        q = 1 - p
        if j >= 1:
            cd[j - 1][0].wait()
            cd[j - 1][1].wait()
        if j + 1 < DKC:
            gd[j + 1] = gather(j + 1, q)
        base = w * (DKC * C) + j * C
        gd[j][0].wait()
        gd[j][1].wait()
        cd[j] = (pltpu.async_copy(bufs[2 * p], zs_out.at[pl.ds(base, C), :], csems[2 * p]),
                 pltpu.async_copy(bufs[2 * p + 1], zd_out.at[pl.ds(base, C), :], csems[2 * p + 1]))
    cd[DKC - 1][0].wait()
    cd[DKC - 1][1].wait()


_BR = 1024


def _dinv_block(d0_blk, d1_blk):
    d = d0_blk + d1_blk + 1.0
    return jax.lax.rsqrt(d)[:, :1]


def _tc1_body(x_ref, w_ref, d0_ref, d1_ref, hp_ref):
    h = jnp.dot(x_ref[...], w_ref[...], preferred_element_type=jnp.float32)
    hp_ref[...] = h * _dinv_block(d0_ref[...], d1_ref[...])


def _tc2_body(a0_ref, a1_ref, hp_ref, d0_ref, d1_ref, b_ref, w_ref, hp2_ref):
    dinv = _dinv_block(d0_ref[...], d1_ref[...])
    z1 = jnp.maximum(
        dinv * (a0_ref[...] + a1_ref[...] + hp_ref[...]) + b_ref[...], 0.0)
    h2 = jnp.dot(z1, w_ref[...], preferred_element_type=jnp.float32)
    hp2_ref[...] = h2 * dinv


def _tc3_body(a0_ref, a1_ref, hp2_ref, d0_ref, d1_ref, b_ref, z_ref):
    dinv = _dinv_block(d0_ref[...], d1_ref[...])
    z_ref[...] = dinv * (a0_ref[...] + a1_ref[...] + hp2_ref[...]) + b_ref[...]


def _tc4_body(zs_ref, zd_ref, o_ref):
    p = (zs_ref[...] * zd_ref[...]).reshape(_BR // 128, 128, D_OUT)
    o_ref[...] = jnp.sum(p, axis=-1)


def _row_spec(d):
    return pl.BlockSpec((_BR, d), lambda i: (i, 0))


_deg_spec = pl.BlockSpec((_BR, 8), lambda i: (i, 0))


def _full(shape):
    return pl.BlockSpec(shape, lambda i: (0,) * len(shape))


_GRID = NP // _BR


def _tc1(x_p, W1, d0, d1):
    return pl.pallas_call(
        _tc1_body,
        grid=(_GRID,),
        in_specs=[_row_spec(D_IN), _full((D_IN, D_H)), _deg_spec, _deg_spec],
        out_specs=_row_spec(D_H),
        out_shape=jax.ShapeDtypeStruct((NP, D_H), jnp.float32),
    )(x_p, W1, d0, d1)


def _tc2(a0, a1, hp, d0, d1, b1, W2):
    return pl.pallas_call(
        _tc2_body,
        grid=(_GRID,),
        in_specs=[_row_spec(D_H), _row_spec(D_H), _row_spec(D_H), _deg_spec,
                  _deg_spec, _full((1, D_H)), _full((D_H, D_OUT))],
        out_specs=_row_spec(D_OUT),
        out_shape=jax.ShapeDtypeStruct((NP, D_OUT), jnp.float32),
    )(a0, a1, hp, d0, d1, b1, W2)


def _tc3(a0, a1, hp2, d0, d1, b2):
    return pl.pallas_call(
        _tc3_body,
        grid=(_GRID,),
        in_specs=[_row_spec(D_OUT), _row_spec(D_OUT), _row_spec(D_OUT),
                  _deg_spec, _deg_spec, _full((1, D_OUT))],
        out_specs=_row_spec(D_OUT),
        out_shape=jax.ShapeDtypeStruct((NP, D_OUT), jnp.float32),
    )(a0, a1, hp2, d0, d1, b2)


def _tc4(zs, zd):
    return pl.pallas_call(
        _tc4_body,
        grid=(ELP // _BR,),
        in_specs=[_row_spec(D_OUT), _row_spec(D_OUT)],
        out_specs=pl.BlockSpec((_BR // 128, 128), lambda i: (i, 0)),
        out_shape=jax.ShapeDtypeStruct((ELP // 128, 128), jnp.float32),
    )(zs, zd)


def kernel(x, edge_index, edge_label_index, W1, b1, W2, b2):
    x_p = jnp.pad(x, ((0, NP - N), (0, 0)))
    src = jnp.concatenate(
        [edge_index[0], jnp.zeros((EP - E,), jnp.int32)]).reshape(NW, SUP, CS, C)
    dst = jnp.concatenate(
        [edge_index[1], jnp.full((EP - E,), N, jnp.int32)]).reshape(NW, SUP, CS, C)
    pad = jnp.zeros((ELP - EL,), jnp.int32)
    sidx = jnp.concatenate([edge_label_index[0], pad]).reshape(NW, DKC, C)
    didx = jnp.concatenate([edge_label_index[1], pad]).reshape(NW, DKC, C)

    deg2 = _sc_degree(dst)
    d0 = jnp.broadcast_to(deg2[0][:, None], (NP, 8))
    d1 = jnp.broadcast_to(deg2[1][:, None], (NP, 8))

    hp = _tc1(x_p, W1, d0, d1)
    acc1 = _sc_agg_128(hp, src, dst)
    hp2 = _tc2(acc1[0], acc1[1], hp, d0, d1, b1.reshape(1, D_H), W2)
    acc2 = _sc_agg_64u(hp2, src, dst)
    z = _tc3(acc2[0], acc2[1], hp2, d0, d1, b2.reshape(1, D_OUT))

    zs, zd = _sc_decode(z, sidx, didx)
    dots = _tc4(zs, zd)
    return dots.reshape(ELP)[:EL]
